# baseline (device time: 99320 ns/iter reference)
import os

import jax
import jax.numpy as jnp
from jax import lax
from jax.experimental import pallas as pl
from jax.experimental.pallas import tpu as pltpu

_PROBE = os.environ.get("KERNEL_PROBE", "")

N_DEV = 4
CM = 512
HL = 1024
NSUB = int(os.environ.get("KERNEL_NSUB", "2"))
SUB = CM // NSUB


def kernel(A, B):
    m, k = A.shape
    _, n = B.shape

    def body(a_ref, b_ref, out_ref,
             a_bf, b_bf, cw_comm, ccw_comm, cw_stage, ccw_stage,
             ag_cw, ag_ccw, cw_ssem, cw_rsem, ccw_ssem, ccw_rsem):
        my = lax.axis_index("i")
        left = lax.rem(my + N_DEV - 1, N_DEV)
        right = lax.rem(my + 1, N_DEV)

        a_bf[:, :] = a_ref[:, :].astype(jnp.bfloat16)
        b_bf[:, :] = b_ref[:, :].astype(jnp.bfloat16)

        barrier_sem = pltpu.get_barrier_semaphore()
        for nbr in (left, right):
            pl.semaphore_signal(
                barrier_sem, inc=1,
                device_id=(nbr,), device_id_type=pl.DeviceIdType.MESH,
            )
        pl.semaphore_wait(barrier_sem, 2)

        def dot(a, b):
            if _PROBE == "nocompute":
                return jnp.zeros((a.shape[0], b.shape[1]), jnp.float32)
            return jnp.dot(a, b, preferred_element_type=jnp.float32)

        def rows(c, s):
            return pl.ds(c * CM + s * SUB, SUB)

        def srows(s):
            return pl.ds(s * SUB, SUB)

        def cols(half):
            return pl.ds(half * HL, HL)

        def cw_sc(h):
            return lax.rem(my + 3 - h, N_DEV)

        def cw_rc(h):
            return lax.rem(my + N_DEV + 2 - h, N_DEV)

        def ccw_sc(h):
            return lax.rem(my + 1 + h, N_DEV)

        def ccw_rc(h):
            return lax.rem(my + 2 + h, N_DEV)

        all_descs = []

        def rs_stage_slot(h, direction):
            stage, comm = (cw_stage, cw_comm) if direction == 0 else (
                ccw_stage, ccw_comm)
            return stage.at[0] if h == 0 else comm.at[h - 1]

        def rs_start(h, s, direction):
            if _PROBE == "nocomm":
                return None
            comm = cw_comm if direction == 0 else ccw_comm
            ssem, rsem = (cw_ssem, cw_rsem) if direction == 0 else (
                ccw_ssem, ccw_rsem)
            tgt = right if direction == 0 else left
            d = pltpu.make_async_remote_copy(
                src_ref=rs_stage_slot(h, direction).at[srows(s), :],
                dst_ref=comm.at[h, srows(s), :],
                send_sem=ssem.at[NSUB * h + s],
                recv_sem=rsem.at[NSUB * h + s],
                device_id=(tgt,), device_id_type=pl.DeviceIdType.MESH,
            )
            d.start()
            all_descs.append(d)
            return d

        def ag_start(h, s, direction):
            if _PROBE == "nocomm":
                return None
            buf = ag_cw if direction == 0 else ag_ccw
            ssem, rsem = (cw_ssem, cw_rsem) if direction == 0 else (
                ccw_ssem, ccw_rsem)
            tgt = right if direction == 0 else left
            d = pltpu.make_async_remote_copy(
                src_ref=buf.at[h, srows(s), :],
                dst_ref=buf.at[h + 1, srows(s), :],
                send_sem=ssem.at[(N_DEV - 1) * NSUB + NSUB * h + s],
                recv_sem=rsem.at[(N_DEV - 1) * NSUB + NSUB * h + s],
                device_id=(tgt,), device_id_type=pl.DeviceIdType.MESH,
            )
            d.start()
            all_descs.append(d)
            return d

        rs_inflight = {}
        for s in range(NSUB):
            v = dot(a_bf[rows(cw_sc(0), s), :], b_bf[:, cols(0)])
            cw_stage[0, srows(s), :] = v.astype(jnp.bfloat16)
            cw_d = rs_start(0, s, 0)
            v = dot(a_bf[rows(ccw_sc(0), s), :], b_bf[:, cols(1)])
            ccw_stage[0, srows(s), :] = v.astype(jnp.bfloat16)
            ccw_d = rs_start(0, s, 1)
            rs_inflight[s] = (cw_d, ccw_d)

        ag_inflight = {}
        for h in range(N_DEV - 1):
            for s in range(NSUB):
                v_cw = dot(a_bf[rows(cw_rc(h), s), :], b_bf[:, cols(0)])
                v_ccw = dot(a_bf[rows(ccw_rc(h), s), :], b_bf[:, cols(1)])

                cw_d, ccw_d = rs_inflight[s]
                cw_d.wait_recv() if cw_d is not None else None
                acc_cw = v_cw + cw_comm[h, srows(s), :].astype(jnp.float32)
                if h < N_DEV - 2:
                    cw_comm[h, srows(s), :] = acc_cw.astype(jnp.bfloat16)
                    new_cw = rs_start(h + 1, s, 0)

                ccw_d.wait_recv() if ccw_d is not None else None if cw_d is not None else None
                acc_ccw = v_ccw + ccw_comm[h, srows(s), :].astype(jnp.float32)
                if h < N_DEV - 2:
                    ccw_comm[h, srows(s), :] = acc_ccw.astype(jnp.bfloat16)
                    rs_inflight[s] = (new_cw, rs_start(h + 1, s, 1))
                else:
                    r_cw = jnp.maximum(acc_cw, 0.0)
                    r_ccw = jnp.maximum(acc_ccw, 0.0)
                    ag_cw[0, srows(s), :] = r_cw.astype(jnp.bfloat16)
                    ag_ccw[0, srows(s), :] = r_ccw.astype(jnp.bfloat16)
                    ag_inflight[s] = (ag_start(0, s, 0), ag_start(0, s, 1))
                    out_ref[rows(my, s), cols(0)] = r_cw
                    out_ref[rows(my, s), cols(1)] = r_ccw

        for h in range(N_DEV - 1):
            for s in range(NSUB):
                cw_d, ccw_d = ag_inflight[s]
                cw_d.wait_recv() if cw_d is not None else None
                if h < N_DEV - 2:
                    new_cw = ag_start(h + 1, s, 0)
                ccw_d.wait_recv() if ccw_d is not None else None if cw_d is not None else None
                if h < N_DEV - 2:
                    ag_inflight[s] = (new_cw, ag_start(h + 1, s, 1))
                out_ref[rows(lax.rem(my + 3 - h, N_DEV), s), cols(0)] = (
                    ag_cw[h + 1, srows(s), :].astype(jnp.float32))
                out_ref[rows(lax.rem(my + 1 + h, N_DEV), s), cols(1)] = (
                    ag_ccw[h + 1, srows(s), :].astype(jnp.float32))

        for d in all_descs:
            d.wait_send()

    n_sems = 2 * (N_DEV - 1) * NSUB
    bf = jnp.bfloat16
    return pl.pallas_call(
        body,
        out_shape=jax.ShapeDtypeStruct((m, n), jnp.float32),
        in_specs=[
            pl.BlockSpec(memory_space=pltpu.VMEM),
            pl.BlockSpec(memory_space=pltpu.VMEM),
        ],
        out_specs=pl.BlockSpec(memory_space=pltpu.VMEM),
        scratch_shapes=[
            pltpu.VMEM((m, k), bf),
            pltpu.VMEM((k, n), bf),
            pltpu.VMEM((N_DEV - 1, CM, HL), bf),
            pltpu.VMEM((N_DEV - 1, CM, HL), bf),
            pltpu.VMEM((1, CM, HL), bf),
            pltpu.VMEM((1, CM, HL), bf),
            pltpu.VMEM((N_DEV, CM, HL), bf),
            pltpu.VMEM((N_DEV, CM, HL), bf),
            pltpu.SemaphoreType.DMA((n_sems,)),
            pltpu.SemaphoreType.DMA((n_sems,)),
            pltpu.SemaphoreType.DMA((n_sems,)),
            pltpu.SemaphoreType.DMA((n_sems,)),
        ],
        compiler_params=pltpu.CompilerParams(
            collective_id=0, vmem_limit_bytes=100 * 1024 * 1024,
        ),
    )(A, B)


# device time: 91610 ns/iter; 1.0842x vs baseline; 1.0842x over previous
import os

import jax
import jax.numpy as jnp
from jax import lax
from jax.experimental import pallas as pl
from jax.experimental.pallas import tpu as pltpu

_PROBE = os.environ.get("KERNEL_PROBE", "")

N_DEV = 4
CM = 512
HL = 1024
NSUB = int(os.environ.get("KERNEL_NSUB", "2"))
SUB = CM // NSUB


def kernel(A, B):
    m, k = A.shape
    _, n = B.shape

    def body(a_ref, b_ref, out_ref,
             a_bf, b_bf, cw_comm, ccw_comm, cw_stage, ccw_stage,
             cw_ssem, cw_rsem, ccw_ssem, ccw_rsem):
        my = lax.axis_index("i")
        left = lax.rem(my + N_DEV - 1, N_DEV)
        right = lax.rem(my + 1, N_DEV)

        a_bf[:, :] = a_ref[:, :].astype(jnp.bfloat16)
        b_bf[:, :] = b_ref[:, :].astype(jnp.bfloat16)

        barrier_sem = pltpu.get_barrier_semaphore()
        for nbr in (left, right):
            pl.semaphore_signal(
                barrier_sem, inc=1,
                device_id=(nbr,), device_id_type=pl.DeviceIdType.MESH,
            )
        pl.semaphore_wait(barrier_sem, 2)

        def dot(a, b):
            if _PROBE == "nocompute":
                return jnp.zeros((a.shape[0], b.shape[1]), jnp.float32)
            return jnp.dot(a, b, preferred_element_type=jnp.float32)

        def rows(c, s):
            return pl.ds(c * CM + s * SUB, SUB)

        def srows(s):
            return pl.ds(s * SUB, SUB)

        def cols(half):
            return pl.ds(half * HL, HL)

        def cw_sc(h):
            return lax.rem(my + 3 - h, N_DEV)

        def cw_rc(h):
            return lax.rem(my + N_DEV + 2 - h, N_DEV)

        def ccw_sc(h):
            return lax.rem(my + 1 + h, N_DEV)

        def ccw_rc(h):
            return lax.rem(my + 2 + h, N_DEV)

        all_descs = []

        def rs_stage_slot(h, direction):
            stage, comm = (cw_stage, cw_comm) if direction == 0 else (
                ccw_stage, ccw_comm)
            return stage.at[0] if h == 0 else comm.at[h - 1]

        def rs_start(h, s, direction):
            if _PROBE == "nocomm":
                return None
            comm = cw_comm if direction == 0 else ccw_comm
            ssem, rsem = (cw_ssem, cw_rsem) if direction == 0 else (
                ccw_ssem, ccw_rsem)
            tgt = right if direction == 0 else left
            d = pltpu.make_async_remote_copy(
                src_ref=rs_stage_slot(h, direction).at[srows(s), :],
                dst_ref=comm.at[h, srows(s), :],
                send_sem=ssem.at[NSUB * h + s],
                recv_sem=rsem.at[NSUB * h + s],
                device_id=(tgt,), device_id_type=pl.DeviceIdType.MESH,
            )
            d.start()
            all_descs.append(d)
            return d

        def ag_start(h, s, direction):
            if _PROBE == "nocomm":
                return None
            if direction == 0:
                c = lax.rem(my + N_DEV - h, N_DEV)
            else:
                c = lax.rem(my + h, N_DEV)
            ssem, rsem = (cw_ssem, cw_rsem) if direction == 0 else (
                ccw_ssem, ccw_rsem)
            tgt = right if direction == 0 else left
            d = pltpu.make_async_remote_copy(
                src_ref=out_ref.at[rows(c, s), cols(direction)],
                dst_ref=out_ref.at[rows(c, s), cols(direction)],
                send_sem=ssem.at[(N_DEV - 1) * NSUB + NSUB * h + s],
                recv_sem=rsem.at[(N_DEV - 1) * NSUB + NSUB * h + s],
                device_id=(tgt,), device_id_type=pl.DeviceIdType.MESH,
            )
            d.start()
            all_descs.append(d)
            return d

        rs_inflight = {}
        for s in range(NSUB):
            v = dot(a_bf[rows(cw_sc(0), s), :], b_bf[:, cols(0)])
            cw_stage[0, srows(s), :] = v.astype(jnp.bfloat16)
            cw_d = rs_start(0, s, 0)
            v = dot(a_bf[rows(ccw_sc(0), s), :], b_bf[:, cols(1)])
            ccw_stage[0, srows(s), :] = v.astype(jnp.bfloat16)
            ccw_d = rs_start(0, s, 1)
            rs_inflight[s] = (cw_d, ccw_d)

        ag_inflight = {}
        for h in range(N_DEV - 1):
            for s in range(NSUB):
                v_cw = dot(a_bf[rows(cw_rc(h), s), :], b_bf[:, cols(0)])
                v_ccw = dot(a_bf[rows(ccw_rc(h), s), :], b_bf[:, cols(1)])

                cw_d, ccw_d = rs_inflight[s]
                cw_d.wait_recv() if cw_d is not None else None
                acc_cw = v_cw + cw_comm[h, srows(s), :].astype(jnp.float32)
                if h < N_DEV - 2:
                    cw_comm[h, srows(s), :] = acc_cw.astype(jnp.bfloat16)
                    new_cw = rs_start(h + 1, s, 0)

                ccw_d.wait_recv() if ccw_d is not None else None
                acc_ccw = v_ccw + ccw_comm[h, srows(s), :].astype(jnp.float32)
                if h < N_DEV - 2:
                    ccw_comm[h, srows(s), :] = acc_ccw.astype(jnp.bfloat16)
                    rs_inflight[s] = (new_cw, rs_start(h + 1, s, 1))
                else:
                    out_ref[rows(my, s), cols(0)] = jnp.maximum(
                        acc_cw, 0.0).astype(jnp.bfloat16)
                    out_ref[rows(my, s), cols(1)] = jnp.maximum(
                        acc_ccw, 0.0).astype(jnp.bfloat16)
                    ag_inflight[s] = (ag_start(0, s, 0), ag_start(0, s, 1))

        for h in range(N_DEV - 1):
            for s in range(NSUB):
                cw_d, ccw_d = ag_inflight[s]
                cw_d.wait_recv() if cw_d is not None else None
                if h < N_DEV - 2:
                    new_cw = ag_start(h + 1, s, 0)
                ccw_d.wait_recv() if ccw_d is not None else None
                if h < N_DEV - 2:
                    ag_inflight[s] = (new_cw, ag_start(h + 1, s, 1))

        for d in all_descs:
            d.wait_send()

    n_sems = 2 * (N_DEV - 1) * NSUB
    bf = jnp.bfloat16
    return pl.pallas_call(
        body,
        out_shape=jax.ShapeDtypeStruct((m, n), jnp.bfloat16),
        in_specs=[
            pl.BlockSpec(memory_space=pltpu.VMEM),
            pl.BlockSpec(memory_space=pltpu.VMEM),
        ],
        out_specs=pl.BlockSpec(memory_space=pltpu.VMEM),
        scratch_shapes=[
            pltpu.VMEM((m, k), bf),
            pltpu.VMEM((k, n), bf),
            pltpu.VMEM((N_DEV - 1, CM, HL), bf),
            pltpu.VMEM((N_DEV - 1, CM, HL), bf),
            pltpu.VMEM((1, CM, HL), bf),
            pltpu.VMEM((1, CM, HL), bf),
            pltpu.SemaphoreType.DMA((n_sems,)),
            pltpu.SemaphoreType.DMA((n_sems,)),
            pltpu.SemaphoreType.DMA((n_sems,)),
            pltpu.SemaphoreType.DMA((n_sems,)),
        ],
        compiler_params=pltpu.CompilerParams(
            collective_id=0, vmem_limit_bytes=100 * 1024 * 1024,
        ),
    )(A, B)
